# speedup vs baseline: 929.5953x; 929.5953x over previous
import jax
import jax.numpy as jnp
from jax.experimental import pallas as pl

_P = 1024
_D = 1024
_TEMP = 0.01
_K = 103


def _body(x_ref, V_ref, alpha_ref, out_ref):
    logits = alpha_ref[:, :] * (1.0 / _TEMP)
    m = jnp.max(logits, axis=0, keepdims=True)
    e = jnp.exp(logits - m)
    s = jnp.sum(e, axis=0, keepdims=True)
    aw = jnp.clip(e * (_K / s), 0.0, 1.0)

    U = V_ref[:, :] * aw

    col = jax.lax.broadcasted_iota(jnp.int32, (_P, _D), 1)
    A = U
    for k in range(10):
        shift = 1 << k
        A = jnp.where((col & shift) != 0, jnp.roll(A, shift, axis=0), A)

    out_ref[:, :] = jax.lax.dot_general(
        x_ref[:, :], A, (((1,), (1,)), ((), ())),
        preferred_element_type=jnp.float32)


@jax.jit
def kernel(x, V, alpha):
    return pl.pallas_call(
        _body,
        out_shape=jax.ShapeDtypeStruct((x.shape[0], _P), x.dtype),
    )(x, V, alpha.reshape(_P, 1))
